# Initial kernel scaffold; baseline (speedup 1.0000x reference)
import functools

import jax
import jax.numpy as jnp
from jax import lax
from jax.experimental import pallas as pl
from jax.experimental.pallas import tpu as pltpu
from jax.experimental.pallas import tpu_sc as plsc

_CHUNK = 128
_NCORES = 2
_NSUB = 16
_NW = _NCORES * _NSUB
_LANES = 16


def _sc_mesh():
    return plsc.VectorSubcoreMesh(core_axis_name="c", subcore_axis_name="s")


def _sc_hist(dst_pad, n_pad):
    e_pad = dst_pad.shape[0]
    cpw = e_pad // (_NW * _CHUNK)
    sl = n_pad // _NSUB

    @functools.partial(
        pl.kernel,
        out_type=jax.ShapeDtypeStruct((_NCORES, n_pad, _LANES), jnp.float32),
        mesh=_sc_mesh(),
        scratch_types=[
            pltpu.VMEM_SHARED((n_pad, _LANES), jnp.float32),
            pltpu.VMEM((_CHUNK,), jnp.int32),
            pltpu.VMEM((_CHUNK, _LANES), jnp.float32),
            pltpu.VMEM((_CHUNK, _LANES), jnp.float32),
        ],
    )
    def hist_kernel(dst_hbm, out_hbm, acc, idx_v, ones_v, zero_v):
        c = lax.axis_index("c")
        s = lax.axis_index("s")
        w = s * _NCORES + c

        @pl.loop(0, _CHUNK)
        def _(r):
            ones_v[r, :] = jnp.ones((_LANES,), jnp.float32)
            zero_v[r, :] = jnp.zeros((_LANES,), jnp.float32)

        @pl.loop(0, sl // _CHUNK)
        def _(b):
            pltpu.sync_copy(zero_v, acc.at[pl.ds(s * sl + b * _CHUNK, _CHUNK)])

        plsc.subcore_barrier()

        @pl.loop(0, cpw)
        def _(k):
            base = (w * cpw + k) * _CHUNK
            pltpu.sync_copy(dst_hbm.at[pl.ds(base, _CHUNK)], idx_v)
            pltpu.sync_copy(ones_v, acc.at[idx_v], add=True)

        plsc.subcore_barrier()

        @pl.loop(0, sl // _CHUNK)
        def _(b):
            r0 = s * sl + b * _CHUNK
            pltpu.sync_copy(acc.at[pl.ds(r0, _CHUNK)],
                            out_hbm.at[c, pl.ds(r0, _CHUNK)])

    return hist_kernel(dst_pad)


def _sc_agg(y_pad, src_pad, dst_pad):
    n_pad, d = y_pad.shape
    e_pad = src_pad.shape[0]
    cpw = e_pad // (_NW * _CHUNK)
    sl = n_pad // _NSUB

    @functools.partial(
        pl.kernel,
        out_type=jax.ShapeDtypeStruct((_NCORES, n_pad, d), jnp.float32),
        mesh=_sc_mesh(),
        scratch_types=[
            pltpu.VMEM_SHARED((n_pad, d), jnp.float32),
            pltpu.VMEM((_CHUNK,), jnp.int32),
            pltpu.VMEM((_CHUNK,), jnp.int32),
            pltpu.VMEM((_CHUNK, d), jnp.float32),
            pltpu.VMEM((_CHUNK, d), jnp.float32),
        ],
    )
    def agg_kernel(y_hbm, src_hbm, dst_hbm, out_hbm,
                   acc, si_v, di_v, rows_v, zero_v):
        c = lax.axis_index("c")
        s = lax.axis_index("s")
        w = s * _NCORES + c

        @pl.loop(0, _CHUNK)
        def _(r):
            @pl.loop(0, d // _LANES)
            def _(q):
                zero_v[r, pl.ds(q * _LANES, _LANES)] = jnp.zeros(
                    (_LANES,), jnp.float32)

        @pl.loop(0, sl // _CHUNK)
        def _(b):
            pltpu.sync_copy(zero_v, acc.at[pl.ds(s * sl + b * _CHUNK, _CHUNK)])

        plsc.subcore_barrier()

        @pl.loop(0, cpw)
        def _(k):
            base = (w * cpw + k) * _CHUNK
            pltpu.sync_copy(src_hbm.at[pl.ds(base, _CHUNK)], si_v)
            pltpu.sync_copy(dst_hbm.at[pl.ds(base, _CHUNK)], di_v)
            pltpu.sync_copy(y_hbm.at[si_v], rows_v)
            pltpu.sync_copy(rows_v, acc.at[di_v], add=True)

        plsc.subcore_barrier()

        @pl.loop(0, sl // _CHUNK)
        def _(b):
            r0 = s * sl + b * _CHUNK
            pltpu.sync_copy(acc.at[pl.ds(r0, _CHUNK)],
                            out_hbm.at[c, pl.ds(r0, _CHUNK)])

    return agg_kernel(y_pad, src_pad, dst_pad)


_BR = 1024


def _tc_matmul(x_pad, w):
    n_pad, din = x_pad.shape
    dout = w.shape[1]

    def body(x_ref, w_ref, o_ref):
        o_ref[...] = jnp.dot(x_ref[...], w_ref[...],
                             preferred_element_type=jnp.float32,
                             precision=lax.Precision.HIGHEST)

    return pl.pallas_call(
        body,
        grid=(n_pad // _BR,),
        in_specs=[pl.BlockSpec((_BR, din), lambda i: (i, 0)),
                  pl.BlockSpec((din, dout), lambda i: (0, 0))],
        out_specs=pl.BlockSpec((_BR, dout), lambda i: (i, 0)),
        out_shape=jax.ShapeDtypeStruct((n_pad, dout), jnp.float32),
    )(x_pad, w)


def _tc_scale(hist, xw, n):
    n_pad, d = xw.shape

    def body(h_ref, xw_ref, y_ref, di_ref):
        i = pl.program_id(0)
        deg = h_ref[0, :, 0:1] + h_ref[1, :, 0:1] + 1.0
        rid = lax.broadcasted_iota(jnp.int32, (_BR, 1), 0) + i * _BR
        dinv = jnp.where(rid < n, lax.rsqrt(deg), 0.0)
        di_ref[...] = dinv
        y_ref[...] = xw_ref[...] * dinv

    return pl.pallas_call(
        body,
        grid=(n_pad // _BR,),
        in_specs=[pl.BlockSpec((_NCORES, _BR, _LANES), lambda i: (0, i, 0)),
                  pl.BlockSpec((_BR, d), lambda i: (i, 0))],
        out_specs=[pl.BlockSpec((_BR, d), lambda i: (i, 0)),
                   pl.BlockSpec((_BR, 1), lambda i: (i, 0))],
        out_shape=[jax.ShapeDtypeStruct((n_pad, d), jnp.float32),
                   jax.ShapeDtypeStruct((n_pad, 1), jnp.float32)],
    )(hist, xw)


def _tc_combine(parts, y, dinv, b, w2):
    n_pad, d = y.shape
    dout = w2.shape[1]

    def body(p_ref, y_ref, di_ref, b_ref, w_ref, o_ref):
        di = di_ref[...]
        h = jnp.maximum(di * (p_ref[0] + p_ref[1] + y_ref[...]) + b_ref[...],
                        0.0)
        o_ref[...] = di * jnp.dot(h, w_ref[...],
                                  preferred_element_type=jnp.float32,
                                  precision=lax.Precision.HIGHEST)

    return pl.pallas_call(
        body,
        grid=(n_pad // _BR,),
        in_specs=[pl.BlockSpec((_NCORES, _BR, d), lambda i: (0, i, 0)),
                  pl.BlockSpec((_BR, d), lambda i: (i, 0)),
                  pl.BlockSpec((_BR, 1), lambda i: (i, 0)),
                  pl.BlockSpec((1, d), lambda i: (0, 0)),
                  pl.BlockSpec((d, dout), lambda i: (0, 0))],
        out_specs=pl.BlockSpec((_BR, dout), lambda i: (i, 0)),
        out_shape=jax.ShapeDtypeStruct((n_pad, dout), jnp.float32),
    )(parts, y, dinv, b, w2)


def _tc_final(parts, y, dinv, b):
    n_pad, d = y.shape

    def body(p_ref, y_ref, di_ref, b_ref, o_ref):
        o_ref[...] = (di_ref[...] * (p_ref[0] + p_ref[1] + y_ref[...])
                      + b_ref[...])

    return pl.pallas_call(
        body,
        grid=(n_pad // _BR,),
        in_specs=[pl.BlockSpec((_NCORES, _BR, d), lambda i: (0, i, 0)),
                  pl.BlockSpec((_BR, d), lambda i: (i, 0)),
                  pl.BlockSpec((_BR, 1), lambda i: (i, 0)),
                  pl.BlockSpec((1, d), lambda i: (0, 0))],
        out_specs=pl.BlockSpec((_BR, d), lambda i: (i, 0)),
        out_shape=jax.ShapeDtypeStruct((n_pad, d), jnp.float32),
    )(parts, y, dinv, b)


def kernel(x, edge_index, W1, b1, W2, b2):
    n, din = x.shape
    e = edge_index.shape[1]

    n_pad = ((n + _NSUB * _CHUNK - 1) // (_NSUB * _CHUNK)) * (_NSUB * _CHUNK)
    step = _NW * _CHUNK * 2
    e_pad = ((e + step - 1) // step) * step

    src = edge_index[0].astype(jnp.int32)
    dst = edge_index[1].astype(jnp.int32)
    fill = jnp.full((e_pad - e,), n, jnp.int32)
    src_pad = jnp.concatenate([src, fill])
    dst_pad = jnp.concatenate([dst, fill])
    x_pad = jnp.pad(x, ((0, n_pad - n), (0, 0)))
    b1r = b1.reshape(1, -1)
    b2r = b2.reshape(1, -1)

    hist = _sc_hist(dst_pad, n_pad)
    xw1 = _tc_matmul(x_pad, W1)
    y1, dinv = _tc_scale(hist, xw1, n)
    parts1 = _sc_agg(y1, src_pad, dst_pad)
    y2 = _tc_combine(parts1, y1, dinv, b1r, W2)
    parts2 = _sc_agg(y2, src_pad, dst_pad)
    out_pad = _tc_final(parts2, y2, dinv, b2r)
    return out_pad[:n]


# speedup vs baseline: 9.4282x; 9.4282x over previous
import functools

import jax
import jax.numpy as jnp
from jax import lax
from jax.experimental import pallas as pl
from jax.experimental.pallas import tpu as pltpu
from jax.experimental.pallas import tpu_sc as plsc

_CHUNK = 128
_NCORES = 2
_NSUB = 16
_NW = _NCORES * _NSUB
_LANES = 16


def _sc_mesh():
    return plsc.VectorSubcoreMesh(core_axis_name="c", subcore_axis_name="s")


def _sc_hist(dst_pad, n_pad):
    e_pad = dst_pad.shape[0]
    cpw = e_pad // (_NW * _CHUNK)
    sl = n_pad // _NSUB

    @functools.partial(
        pl.kernel,
        out_type=jax.ShapeDtypeStruct((_NCORES, n_pad, _LANES), jnp.float32),
        mesh=_sc_mesh(),
        scratch_types=[
            pltpu.VMEM_SHARED((n_pad, _LANES), jnp.float32),
            pltpu.VMEM((_CHUNK,), jnp.int32),
            pltpu.VMEM((_CHUNK, _LANES), jnp.float32),
            pltpu.VMEM((_CHUNK, _LANES), jnp.float32),
        ],
    )
    def hist_kernel(dst_hbm, out_hbm, acc, idx_v, ones_v, zero_v):
        c = lax.axis_index("c")
        s = lax.axis_index("s")
        w = s * _NCORES + c

        @pl.loop(0, _CHUNK)
        def _(r):
            ones_v[r, :] = jnp.ones((_LANES,), jnp.float32)
            zero_v[r, :] = jnp.zeros((_LANES,), jnp.float32)

        @pl.loop(0, sl // _CHUNK)
        def _(b):
            pltpu.sync_copy(zero_v, acc.at[pl.ds(s * sl + b * _CHUNK, _CHUNK)])

        plsc.subcore_barrier()

        @pl.loop(0, cpw)
        def _(k):
            base = (w * cpw + k) * _CHUNK
            pltpu.sync_copy(dst_hbm.at[pl.ds(base, _CHUNK)], idx_v)
            pltpu.sync_copy(ones_v, acc.at[idx_v], add=True)

        plsc.subcore_barrier()

        @pl.loop(0, sl // _CHUNK)
        def _(b):
            r0 = s * sl + b * _CHUNK
            pltpu.sync_copy(acc.at[pl.ds(r0, _CHUNK)],
                            out_hbm.at[c, pl.ds(r0, _CHUNK)])

    return hist_kernel(dst_pad)


def _sc_agg(y_pad, src_pad, dst_pad):
    n_pad, d = y_pad.shape
    e_pad = src_pad.shape[0]
    cpw = e_pad // (_NW * _CHUNK)
    sl = n_pad // _NSUB

    @functools.partial(
        pl.kernel,
        out_type=jax.ShapeDtypeStruct((_NCORES, n_pad, d), jnp.float32),
        mesh=_sc_mesh(),
        compiler_params=pltpu.CompilerParams(use_tc_tiling_on_sc=False),
        scratch_types=[
            pltpu.VMEM_SHARED((n_pad, d), jnp.float32),
            pltpu.VMEM((_CHUNK,), jnp.int32),
            pltpu.VMEM((_CHUNK,), jnp.int32),
            pltpu.VMEM((_CHUNK, d), jnp.float32),
            pltpu.VMEM((_CHUNK, d), jnp.float32),
        ],
    )
    def agg_kernel(y_hbm, src_hbm, dst_hbm, out_hbm,
                   acc, si_v, di_v, rows_v, zero_v):
        c = lax.axis_index("c")
        s = lax.axis_index("s")
        w = s * _NCORES + c

        @pl.loop(0, _CHUNK)
        def _(r):
            @pl.loop(0, d // _LANES)
            def _(q):
                zero_v[r, pl.ds(q * _LANES, _LANES)] = jnp.zeros(
                    (_LANES,), jnp.float32)

        @pl.loop(0, sl // _CHUNK)
        def _(b):
            pltpu.sync_copy(zero_v, acc.at[pl.ds(s * sl + b * _CHUNK, _CHUNK)])

        plsc.subcore_barrier()

        @pl.loop(0, cpw)
        def _(k):
            base = (w * cpw + k) * _CHUNK
            pltpu.sync_copy(src_hbm.at[pl.ds(base, _CHUNK)], si_v)
            pltpu.sync_copy(dst_hbm.at[pl.ds(base, _CHUNK)], di_v)
            pltpu.sync_copy(y_hbm.at[si_v], rows_v)
            pltpu.sync_copy(rows_v, acc.at[di_v], add=True)

        plsc.subcore_barrier()

        @pl.loop(0, sl // _CHUNK)
        def _(b):
            r0 = s * sl + b * _CHUNK
            pltpu.sync_copy(acc.at[pl.ds(r0, _CHUNK)],
                            out_hbm.at[c, pl.ds(r0, _CHUNK)])

    return agg_kernel(y_pad, src_pad, dst_pad)


_BR = 1024


def _tc_matmul(x_pad, w):
    n_pad, din = x_pad.shape
    dout = w.shape[1]

    def body(x_ref, w_ref, o_ref):
        o_ref[...] = jnp.dot(x_ref[...], w_ref[...],
                             preferred_element_type=jnp.float32,
                             precision=lax.Precision.HIGHEST)

    return pl.pallas_call(
        body,
        grid=(n_pad // _BR,),
        in_specs=[pl.BlockSpec((_BR, din), lambda i: (i, 0)),
                  pl.BlockSpec((din, dout), lambda i: (0, 0))],
        out_specs=pl.BlockSpec((_BR, dout), lambda i: (i, 0)),
        out_shape=jax.ShapeDtypeStruct((n_pad, dout), jnp.float32),
    )(x_pad, w)


def _tc_scale(hist, xw, n):
    n_pad, d = xw.shape

    def body(h_ref, xw_ref, y_ref, di_ref):
        i = pl.program_id(0)
        deg = h_ref[0, :, 0:1] + h_ref[1, :, 0:1] + 1.0
        rid = lax.broadcasted_iota(jnp.int32, (_BR, 1), 0) + i * _BR
        dinv = jnp.where(rid < n, lax.rsqrt(deg), 0.0)
        di_ref[...] = dinv
        y_ref[...] = xw_ref[...] * dinv

    return pl.pallas_call(
        body,
        grid=(n_pad // _BR,),
        in_specs=[pl.BlockSpec((_NCORES, _BR, _LANES), lambda i: (0, i, 0)),
                  pl.BlockSpec((_BR, d), lambda i: (i, 0))],
        out_specs=[pl.BlockSpec((_BR, d), lambda i: (i, 0)),
                   pl.BlockSpec((_BR, 1), lambda i: (i, 0))],
        out_shape=[jax.ShapeDtypeStruct((n_pad, d), jnp.float32),
                   jax.ShapeDtypeStruct((n_pad, 1), jnp.float32)],
    )(hist, xw)


def _tc_combine(parts, y, dinv, b, w2):
    n_pad, d = y.shape
    dout = w2.shape[1]

    def body(p_ref, y_ref, di_ref, b_ref, w_ref, o_ref):
        di = di_ref[...]
        h = jnp.maximum(di * (p_ref[0] + p_ref[1] + y_ref[...]) + b_ref[...],
                        0.0)
        o_ref[...] = di * jnp.dot(h, w_ref[...],
                                  preferred_element_type=jnp.float32,
                                  precision=lax.Precision.HIGHEST)

    return pl.pallas_call(
        body,
        grid=(n_pad // _BR,),
        in_specs=[pl.BlockSpec((_NCORES, _BR, d), lambda i: (0, i, 0)),
                  pl.BlockSpec((_BR, d), lambda i: (i, 0)),
                  pl.BlockSpec((_BR, 1), lambda i: (i, 0)),
                  pl.BlockSpec((1, d), lambda i: (0, 0)),
                  pl.BlockSpec((d, dout), lambda i: (0, 0))],
        out_specs=pl.BlockSpec((_BR, dout), lambda i: (i, 0)),
        out_shape=jax.ShapeDtypeStruct((n_pad, dout), jnp.float32),
    )(parts, y, dinv, b, w2)


def _tc_final(parts, y, dinv, b):
    n_pad, d = y.shape

    def body(p_ref, y_ref, di_ref, b_ref, o_ref):
        o_ref[...] = (di_ref[...] * (p_ref[0] + p_ref[1] + y_ref[...])
                      + b_ref[...])

    return pl.pallas_call(
        body,
        grid=(n_pad // _BR,),
        in_specs=[pl.BlockSpec((_NCORES, _BR, d), lambda i: (0, i, 0)),
                  pl.BlockSpec((_BR, d), lambda i: (i, 0)),
                  pl.BlockSpec((_BR, 1), lambda i: (i, 0)),
                  pl.BlockSpec((1, d), lambda i: (0, 0))],
        out_specs=pl.BlockSpec((_BR, d), lambda i: (i, 0)),
        out_shape=jax.ShapeDtypeStruct((n_pad, d), jnp.float32),
    )(parts, y, dinv, b)


def kernel(x, edge_index, W1, b1, W2, b2):
    n, din = x.shape
    e = edge_index.shape[1]

    n_pad = ((n + _NSUB * _CHUNK - 1) // (_NSUB * _CHUNK)) * (_NSUB * _CHUNK)
    step = _NW * _CHUNK * 2
    e_pad = ((e + step - 1) // step) * step

    src = edge_index[0].astype(jnp.int32)
    dst = edge_index[1].astype(jnp.int32)
    fill = jnp.full((e_pad - e,), n, jnp.int32)
    src_pad = jnp.concatenate([src, fill])
    dst_pad = jnp.concatenate([dst, fill])
    x_pad = jnp.pad(x, ((0, n_pad - n), (0, 0)))
    b1r = b1.reshape(1, -1)
    b2r = b2.reshape(1, -1)

    hist = _sc_hist(dst_pad, n_pad)
    xw1 = _tc_matmul(x_pad, W1)
    y1, dinv = _tc_scale(hist, xw1, n)
    parts1 = _sc_agg(y1, src_pad, dst_pad)
    y2 = _tc_combine(parts1, y1, dinv, b1r, W2)
    parts2 = _sc_agg(y2, src_pad, dst_pad)
    out_pad = _tc_final(parts2, y2, dinv, b2r)
    return out_pad[:n]


# speedup vs baseline: 27.0278x; 2.8667x over previous
import functools

import jax
import jax.numpy as jnp
from jax import lax
from jax.experimental import pallas as pl
from jax.experimental.pallas import tpu as pltpu
from jax.experimental.pallas import tpu_sc as plsc

_CHUNK = 128
_NCORES = 2
_NSUB = 16
_NW = _NCORES * _NSUB
_LANES = 16


def _sc_mesh():
    return plsc.VectorSubcoreMesh(core_axis_name="c", subcore_axis_name="s")


def _sc_hist(dst2d, n_pad):
    nchunks = dst2d.shape[0]
    cpw = nchunks // _NW
    sl = n_pad // _NSUB

    @functools.partial(
        pl.kernel,
        out_type=jax.ShapeDtypeStruct((_NCORES, n_pad, _LANES), jnp.float32),
        mesh=_sc_mesh(),
        scratch_types=[
            pltpu.VMEM_SHARED((n_pad, _LANES), jnp.float32),
            pltpu.VMEM((_CHUNK,), jnp.int32),
            pltpu.VMEM((_CHUNK, _LANES), jnp.float32),
            pltpu.VMEM((_CHUNK, _LANES), jnp.float32),
        ],
    )
    def hist_kernel(dst_hbm, out_hbm, acc, idx_v, ones_v, zero_v):
        c = lax.axis_index("c")
        s = lax.axis_index("s")
        w = s * _NCORES + c

        @pl.loop(0, _CHUNK)
        def _(r):
            ones_v[r, :] = jnp.ones((_LANES,), jnp.float32)
            zero_v[r, :] = jnp.zeros((_LANES,), jnp.float32)

        @pl.loop(0, sl // _CHUNK)
        def _(b):
            pltpu.sync_copy(zero_v, acc.at[pl.ds(s * sl + b * _CHUNK, _CHUNK)])

        plsc.subcore_barrier()

        @pl.loop(0, cpw)
        def _(k):
            pltpu.sync_copy(dst_hbm.at[w * cpw + k], idx_v)
            pltpu.sync_copy(ones_v, acc.at[idx_v], add=True)

        plsc.subcore_barrier()

        @pl.loop(0, sl // _CHUNK)
        def _(b):
            r0 = s * sl + b * _CHUNK
            pltpu.sync_copy(acc.at[pl.ds(r0, _CHUNK)],
                            out_hbm.at[c, pl.ds(r0, _CHUNK)])

    return hist_kernel(dst2d)


def _sc_agg(y_pad, src2d, dst2d):
    n_pad, d = y_pad.shape
    nchunks = src2d.shape[0]
    cpw = nchunks // _NW
    sl = n_pad // _NSUB
    assert cpw % 2 == 0 and cpw >= 4

    @functools.partial(
        pl.kernel,
        out_type=jax.ShapeDtypeStruct((_NCORES, n_pad, d), jnp.float32),
        mesh=_sc_mesh(),
        compiler_params=pltpu.CompilerParams(use_tc_tiling_on_sc=False),
        scratch_types=[
            pltpu.VMEM_SHARED((n_pad, d), jnp.float32),
            pltpu.VMEM((_CHUNK,), jnp.int32),
            pltpu.VMEM((_CHUNK,), jnp.int32),
            pltpu.VMEM((_CHUNK,), jnp.int32),
            pltpu.VMEM((_CHUNK,), jnp.int32),
            pltpu.VMEM((_CHUNK, d), jnp.float32),
            pltpu.VMEM((_CHUNK, d), jnp.float32),
            pltpu.VMEM((_LANES, d), jnp.float32),
            pltpu.SemaphoreType.DMA,
            pltpu.SemaphoreType.DMA,
            pltpu.SemaphoreType.DMA,
            pltpu.SemaphoreType.DMA,
        ],
    )
    def agg_kernel(y_hbm, src_hbm, dst_hbm, out_hbm, acc,
                   si0, si1, di0, di1, rows0, rows1, zero_v,
                   g0, g1, i0, i1):
        c = lax.axis_index("c")
        s = lax.axis_index("s")
        w = s * _NCORES + c
        first = w * cpw

        @pl.loop(0, _LANES)
        def _(r):
            @pl.loop(0, d // _LANES)
            def _(q):
                zero_v[r, pl.ds(q * _LANES, _LANES)] = jnp.zeros(
                    (_LANES,), jnp.float32)

        @pl.loop(0, sl // _LANES)
        def _(b):
            pltpu.sync_copy(zero_v, acc.at[pl.ds(s * sl + b * _LANES, _LANES)])

        plsc.subcore_barrier()

        def start_idx(k, si, di, sem):
            pltpu.async_copy(src_hbm.at[first + k], si, sem)
            pltpu.async_copy(dst_hbm.at[first + k], di, sem)

        def wait_idx(k, si, di, sem):
            pltpu.make_async_copy(src_hbm.at[first + k], si, sem).wait()
            pltpu.make_async_copy(dst_hbm.at[first + k], di, sem).wait()

        def body(k, sa, da, ra, ga, ia, sb, db, rb, gb, ib):
            pltpu.make_async_copy(y_hbm.at[sa], ra, ga).wait()
            wait_idx(k + 1, sb, db, ib)
            pltpu.async_copy(y_hbm.at[sb], rb, gb)
            pltpu.sync_copy(ra, acc.at[da], add=True)
            start_idx(k + 2, sa, da, ia)

        pltpu.sync_copy(src_hbm.at[first], si0)
        pltpu.sync_copy(dst_hbm.at[first], di0)
        pltpu.async_copy(y_hbm.at[si0], rows0, g0)
        start_idx(1, si1, di1, i1)

        @pl.loop(0, cpw - 2, step=2)
        def _(k):
            body(k, si0, di0, rows0, g0, i0, si1, di1, rows1, g1, i1)
            body(k + 1, si1, di1, rows1, g1, i1, si0, di0, rows0, g0, i0)

        pltpu.make_async_copy(y_hbm.at[si0], rows0, g0).wait()
        wait_idx(cpw - 1, si1, di1, i1)
        pltpu.async_copy(y_hbm.at[si1], rows1, g1)
        pltpu.sync_copy(rows0, acc.at[di0], add=True)
        pltpu.make_async_copy(y_hbm.at[si1], rows1, g1).wait()
        pltpu.sync_copy(rows1, acc.at[di1], add=True)

        plsc.subcore_barrier()

        @pl.loop(0, sl // _CHUNK)
        def _(b):
            r0 = s * sl + b * _CHUNK
            pltpu.sync_copy(acc.at[pl.ds(r0, _CHUNK)],
                            out_hbm.at[c, pl.ds(r0, _CHUNK)])

    return agg_kernel(y_pad, src2d, dst2d)


_BR = 1024


def _tc_matmul(x_pad, w):
    n_pad, din = x_pad.shape
    dout = w.shape[1]

    def body(x_ref, w_ref, o_ref):
        o_ref[...] = jnp.dot(x_ref[...], w_ref[...],
                             preferred_element_type=jnp.float32,
                             precision=lax.Precision.HIGHEST)

    return pl.pallas_call(
        body,
        grid=(n_pad // _BR,),
        in_specs=[pl.BlockSpec((_BR, din), lambda i: (i, 0)),
                  pl.BlockSpec((din, dout), lambda i: (0, 0))],
        out_specs=pl.BlockSpec((_BR, dout), lambda i: (i, 0)),
        out_shape=jax.ShapeDtypeStruct((n_pad, dout), jnp.float32),
    )(x_pad, w)


def _tc_scale(hist, xw, n):
    n_pad, d = xw.shape

    def body(h_ref, xw_ref, y_ref, di_ref):
        i = pl.program_id(0)
        deg = h_ref[0, :, 0:1] + h_ref[1, :, 0:1] + 1.0
        rid = lax.broadcasted_iota(jnp.int32, (_BR, 1), 0) + i * _BR
        dinv = jnp.where(rid < n, lax.rsqrt(deg), 0.0)
        di_ref[...] = dinv
        y_ref[...] = xw_ref[...] * dinv

    return pl.pallas_call(
        body,
        grid=(n_pad // _BR,),
        in_specs=[pl.BlockSpec((_NCORES, _BR, _LANES), lambda i: (0, i, 0)),
                  pl.BlockSpec((_BR, d), lambda i: (i, 0))],
        out_specs=[pl.BlockSpec((_BR, d), lambda i: (i, 0)),
                   pl.BlockSpec((_BR, 1), lambda i: (i, 0))],
        out_shape=[jax.ShapeDtypeStruct((n_pad, d), jnp.float32),
                   jax.ShapeDtypeStruct((n_pad, 1), jnp.float32)],
    )(hist, xw)


def _tc_combine(parts, y, dinv, b, w2):
    n_pad, d = y.shape
    dout = w2.shape[1]

    def body(p_ref, y_ref, di_ref, b_ref, w_ref, o_ref):
        di = di_ref[...]
        h = jnp.maximum(di * (p_ref[0] + p_ref[1] + y_ref[...]) + b_ref[...],
                        0.0)
        o_ref[...] = di * jnp.dot(h, w_ref[...],
                                  preferred_element_type=jnp.float32,
                                  precision=lax.Precision.HIGHEST)

    return pl.pallas_call(
        body,
        grid=(n_pad // _BR,),
        in_specs=[pl.BlockSpec((_NCORES, _BR, d), lambda i: (0, i, 0)),
                  pl.BlockSpec((_BR, d), lambda i: (i, 0)),
                  pl.BlockSpec((_BR, 1), lambda i: (i, 0)),
                  pl.BlockSpec((1, d), lambda i: (0, 0)),
                  pl.BlockSpec((d, dout), lambda i: (0, 0))],
        out_specs=pl.BlockSpec((_BR, dout), lambda i: (i, 0)),
        out_shape=jax.ShapeDtypeStruct((n_pad, dout), jnp.float32),
    )(parts, y, dinv, b, w2)


def _tc_final(parts, y, dinv, b):
    n_pad, d = y.shape

    def body(p_ref, y_ref, di_ref, b_ref, o_ref):
        o_ref[...] = (di_ref[...] * (p_ref[0] + p_ref[1] + y_ref[...])
                      + b_ref[...])

    return pl.pallas_call(
        body,
        grid=(n_pad // _BR,),
        in_specs=[pl.BlockSpec((_NCORES, _BR, d), lambda i: (0, i, 0)),
                  pl.BlockSpec((_BR, d), lambda i: (i, 0)),
                  pl.BlockSpec((_BR, 1), lambda i: (i, 0)),
                  pl.BlockSpec((1, d), lambda i: (0, 0))],
        out_specs=pl.BlockSpec((_BR, d), lambda i: (i, 0)),
        out_shape=jax.ShapeDtypeStruct((n_pad, d), jnp.float32),
    )(parts, y, dinv, b)


def kernel(x, edge_index, W1, b1, W2, b2):
    n, din = x.shape
    e = edge_index.shape[1]

    n_pad = ((n + _NSUB * _CHUNK - 1) // (_NSUB * _CHUNK)) * (_NSUB * _CHUNK)
    step = _NW * _CHUNK * 2
    e_pad = ((e + step - 1) // step) * step

    src = edge_index[0].astype(jnp.int32)
    dst = edge_index[1].astype(jnp.int32)
    spare = n_pad - n
    fill = n + jnp.arange(e_pad - e, dtype=jnp.int32) % spare
    src2d = jnp.concatenate([src, fill]).reshape(-1, _CHUNK)
    dst2d = jnp.concatenate([dst, fill]).reshape(-1, _CHUNK)
    x_pad = jnp.pad(x, ((0, n_pad - n), (0, 0)))
    b1r = b1.reshape(1, -1)
    b2r = b2.reshape(1, -1)

    hist = _sc_hist(dst2d, n_pad)
    xw1 = _tc_matmul(x_pad, W1)
    y1, dinv = _tc_scale(hist, xw1, n)
    parts1 = _sc_agg(y1, src2d, dst2d)
    y2 = _tc_combine(parts1, y1, dinv, b1r, W2)
    parts2 = _sc_agg(y2, src2d, dst2d)
    out_pad = _tc_final(parts2, y2, dinv, b2r)
    return out_pad[:n]


# speedup vs baseline: 28.9629x; 1.0716x over previous
import functools

import jax
import jax.numpy as jnp
from jax import lax
from jax.experimental import pallas as pl
from jax.experimental.pallas import tpu as pltpu
from jax.experimental.pallas import tpu_sc as plsc

_CHUNK = 128
_NCORES = 2
_NSUB = 16
_NW = _NCORES * _NSUB
_LANES = 16


def _sc_mesh():
    return plsc.VectorSubcoreMesh(core_axis_name="c", subcore_axis_name="s")


def _sc_hist(dst2d, n_pad):
    nchunks = dst2d.shape[0]
    cpw = nchunks // _NW
    sl = n_pad // _NSUB

    @functools.partial(
        pl.kernel,
        out_type=jax.ShapeDtypeStruct((_NCORES, n_pad, _LANES), jnp.float32),
        mesh=_sc_mesh(),
        scratch_types=[
            pltpu.VMEM_SHARED((n_pad, _LANES), jnp.float32),
            pltpu.VMEM((_CHUNK,), jnp.int32),
            pltpu.VMEM((_CHUNK,), jnp.int32),
            pltpu.VMEM((_CHUNK, _LANES), jnp.float32),
            pltpu.VMEM((_CHUNK, _LANES), jnp.float32),
            pltpu.SemaphoreType.DMA,
            pltpu.SemaphoreType.DMA,
        ],
    )
    def hist_kernel(dst_hbm, out_hbm, acc, idx_v, idx_w, ones_v, zero_v,
                    s0, s1):
        c = lax.axis_index("c")
        s = lax.axis_index("s")
        w = s * _NCORES + c

        @pl.loop(0, _CHUNK)
        def _(r):
            ones_v[r, :] = jnp.ones((_LANES,), jnp.float32)
            zero_v[r, :] = jnp.zeros((_LANES,), jnp.float32)

        @pl.loop(0, sl // _CHUNK)
        def _(b):
            pltpu.sync_copy(zero_v, acc.at[pl.ds(s * sl + b * _CHUNK, _CHUNK)])

        plsc.subcore_barrier()

        def load_idx(k, buf, sem):
            pltpu.async_copy(dst_hbm.at[w * cpw + k], buf, sem)

        def wait_idx(k, buf, sem):
            pltpu.make_async_copy(dst_hbm.at[w * cpw + k], buf, sem).wait()

        load_idx(0, idx_v, s0)
        load_idx(1, idx_w, s1)

        @pl.loop(0, cpw, step=2)
        def _(k):
            wait_idx(k, idx_v, s0)
            pltpu.sync_copy(ones_v, acc.at[idx_v], add=True)

            @pl.when(k + 2 < cpw)
            def _():
                load_idx(k + 2, idx_v, s0)

            wait_idx(k + 1, idx_w, s1)
            pltpu.sync_copy(ones_v, acc.at[idx_w], add=True)

            @pl.when(k + 3 < cpw)
            def _():
                load_idx(k + 3, idx_w, s1)

        plsc.subcore_barrier()

        @pl.loop(0, sl // _CHUNK)
        def _(b):
            r0 = s * sl + b * _CHUNK
            pltpu.sync_copy(acc.at[pl.ds(r0, _CHUNK)],
                            out_hbm.at[c, pl.ds(r0, _CHUNK)])

    return hist_kernel(dst2d)


def _sc_agg(y_pad, src2d, dst2d):
    n_pad, d = y_pad.shape
    nchunks = src2d.shape[0]
    cpw = nchunks // _NW
    sl = n_pad // _NSUB
    nbuf = 2
    gdist = nbuf - 1
    assert cpw % nbuf == 0 and cpw >= 2 * nbuf

    @functools.partial(
        pl.kernel,
        out_type=jax.ShapeDtypeStruct((_NCORES, n_pad, d), jnp.float32),
        mesh=_sc_mesh(),
        compiler_params=pltpu.CompilerParams(use_tc_tiling_on_sc=False),
        scratch_types=[
            pltpu.VMEM_SHARED((n_pad, d), jnp.float32),
            [pltpu.VMEM((_CHUNK,), jnp.int32)] * nbuf,
            [pltpu.VMEM((_CHUNK,), jnp.int32)] * nbuf,
            [pltpu.VMEM((_CHUNK, d), jnp.float32)] * nbuf,
            pltpu.VMEM((_LANES, d), jnp.float32),
            [pltpu.SemaphoreType.DMA] * nbuf,
            [pltpu.SemaphoreType.DMA] * nbuf,
        ],
    )
    def agg_kernel(y_hbm, src_hbm, dst_hbm, out_hbm, acc,
                   si, di, rows, zero_v, gsem, isem):
        c = lax.axis_index("c")
        s = lax.axis_index("s")
        w = s * _NCORES + c
        first = w * cpw

        @pl.loop(0, _LANES)
        def _(r):
            @pl.loop(0, d // _LANES)
            def _(q):
                zero_v[r, pl.ds(q * _LANES, _LANES)] = jnp.zeros(
                    (_LANES,), jnp.float32)

        @pl.loop(0, sl // _LANES)
        def _(b):
            pltpu.sync_copy(zero_v, acc.at[pl.ds(s * sl + b * _LANES, _LANES)])

        plsc.subcore_barrier()

        def start_idx(k, b):
            pltpu.async_copy(src_hbm.at[first + k], si[b], isem[b])
            pltpu.async_copy(dst_hbm.at[first + k], di[b], isem[b])

        def wait_idx(k, b):
            pltpu.make_async_copy(src_hbm.at[first + k], si[b], isem[b]).wait()
            pltpu.make_async_copy(dst_hbm.at[first + k], di[b], isem[b]).wait()

        def start_gather(b):
            pltpu.async_copy(y_hbm.at[si[b]], rows[b], gsem[b])

        def wait_gather(b):
            pltpu.make_async_copy(y_hbm.at[si[b]], rows[b], gsem[b]).wait()

        for j in range(nbuf):
            start_idx(j, j)
        for j in range(gdist):
            wait_idx(j, j)
            start_gather(j)

        @pl.loop(0, cpw, step=nbuf)
        def _(k0):
            for j in range(nbuf):
                k = k0 + j
                b = j
                pg = (j + gdist) % nbuf
                wait_gather(b)

                @pl.when(k + gdist < cpw)
                def _():
                    wait_idx(k + gdist, pg)
                    start_gather(pg)

                pltpu.sync_copy(rows[b], acc.at[di[b]], add=True)

                @pl.when(k + nbuf < cpw)
                def _():
                    start_idx(k + nbuf, b)

        plsc.subcore_barrier()

        @pl.loop(0, sl // _CHUNK)
        def _(b):
            r0 = s * sl + b * _CHUNK
            pltpu.sync_copy(acc.at[pl.ds(r0, _CHUNK)],
                            out_hbm.at[c, pl.ds(r0, _CHUNK)])

    return agg_kernel(y_pad, src2d, dst2d)


_BR = 1024


def _tc_matmul(x_pad, w):
    n_pad, din = x_pad.shape
    dout = w.shape[1]

    def body(x_ref, w_ref, o_ref):
        o_ref[...] = jnp.dot(x_ref[...], w_ref[...],
                             preferred_element_type=jnp.float32,
                             precision=lax.Precision.HIGHEST)

    return pl.pallas_call(
        body,
        grid=(n_pad // _BR,),
        in_specs=[pl.BlockSpec((_BR, din), lambda i: (i, 0)),
                  pl.BlockSpec((din, dout), lambda i: (0, 0))],
        out_specs=pl.BlockSpec((_BR, dout), lambda i: (i, 0)),
        out_shape=jax.ShapeDtypeStruct((n_pad, dout), jnp.float32),
    )(x_pad, w)


def _tc_scale(hist, xw, n):
    n_pad, d = xw.shape

    def body(h_ref, xw_ref, y_ref, di_ref):
        i = pl.program_id(0)
        deg = h_ref[0, :, 0:1] + h_ref[1, :, 0:1] + 1.0
        rid = lax.broadcasted_iota(jnp.int32, (_BR, 1), 0) + i * _BR
        dinv = jnp.where(rid < n, lax.rsqrt(deg), 0.0)
        di_ref[...] = dinv
        y_ref[...] = xw_ref[...] * dinv

    return pl.pallas_call(
        body,
        grid=(n_pad // _BR,),
        in_specs=[pl.BlockSpec((_NCORES, _BR, _LANES), lambda i: (0, i, 0)),
                  pl.BlockSpec((_BR, d), lambda i: (i, 0))],
        out_specs=[pl.BlockSpec((_BR, d), lambda i: (i, 0)),
                   pl.BlockSpec((_BR, 1), lambda i: (i, 0))],
        out_shape=[jax.ShapeDtypeStruct((n_pad, d), jnp.float32),
                   jax.ShapeDtypeStruct((n_pad, 1), jnp.float32)],
    )(hist, xw)


def _tc_combine(parts, y, dinv, b, w2):
    n_pad, d = y.shape
    dout = w2.shape[1]

    def body(p_ref, y_ref, di_ref, b_ref, w_ref, o_ref):
        di = di_ref[...]
        h = jnp.maximum(di * (p_ref[0] + p_ref[1] + y_ref[...]) + b_ref[...],
                        0.0)
        o_ref[...] = di * jnp.dot(h, w_ref[...],
                                  preferred_element_type=jnp.float32,
                                  precision=lax.Precision.HIGHEST)

    return pl.pallas_call(
        body,
        grid=(n_pad // _BR,),
        in_specs=[pl.BlockSpec((_NCORES, _BR, d), lambda i: (0, i, 0)),
                  pl.BlockSpec((_BR, d), lambda i: (i, 0)),
                  pl.BlockSpec((_BR, 1), lambda i: (i, 0)),
                  pl.BlockSpec((1, d), lambda i: (0, 0)),
                  pl.BlockSpec((d, dout), lambda i: (0, 0))],
        out_specs=pl.BlockSpec((_BR, dout), lambda i: (i, 0)),
        out_shape=jax.ShapeDtypeStruct((n_pad, dout), jnp.float32),
    )(parts, y, dinv, b, w2)


def _tc_final(parts, y, dinv, b):
    n_pad, d = y.shape

    def body(p_ref, y_ref, di_ref, b_ref, o_ref):
        o_ref[...] = (di_ref[...] * (p_ref[0] + p_ref[1] + y_ref[...])
                      + b_ref[...])

    return pl.pallas_call(
        body,
        grid=(n_pad // _BR,),
        in_specs=[pl.BlockSpec((_NCORES, _BR, d), lambda i: (0, i, 0)),
                  pl.BlockSpec((_BR, d), lambda i: (i, 0)),
                  pl.BlockSpec((_BR, 1), lambda i: (i, 0)),
                  pl.BlockSpec((1, d), lambda i: (0, 0))],
        out_specs=pl.BlockSpec((_BR, d), lambda i: (i, 0)),
        out_shape=jax.ShapeDtypeStruct((n_pad, d), jnp.float32),
    )(parts, y, dinv, b)


def kernel(x, edge_index, W1, b1, W2, b2):
    n, din = x.shape
    e = edge_index.shape[1]

    n_pad = ((n + _NSUB * _CHUNK - 1) // (_NSUB * _CHUNK)) * (_NSUB * _CHUNK)
    step = _NW * _CHUNK * 2
    e_pad = ((e + step - 1) // step) * step

    src = edge_index[0].astype(jnp.int32)
    dst = edge_index[1].astype(jnp.int32)
    spare = n_pad - n
    fill = n + jnp.arange(e_pad - e, dtype=jnp.int32) % spare
    src2d = jnp.concatenate([src, fill]).reshape(-1, _CHUNK)
    dst2d = jnp.concatenate([dst, fill]).reshape(-1, _CHUNK)
    x_pad = jnp.pad(x, ((0, n_pad - n), (0, 0)))
    b1r = b1.reshape(1, -1)
    b2r = b2.reshape(1, -1)

    hist = _sc_hist(dst2d, n_pad)
    xw1 = _tc_matmul(x_pad, W1)
    y1, dinv = _tc_scale(hist, xw1, n)
    parts1 = _sc_agg(y1, src2d, dst2d)
    y2 = _tc_combine(parts1, y1, dinv, b1r, W2)
    parts2 = _sc_agg(y2, src2d, dst2d)
    out_pad = _tc_final(parts2, y2, dinv, b2r)
    return out_pad[:n]


# speedup vs baseline: 29.2813x; 1.0110x over previous
import functools

import jax
import jax.numpy as jnp
from jax import lax
from jax.experimental import pallas as pl
from jax.experimental.pallas import tpu as pltpu
from jax.experimental.pallas import tpu_sc as plsc

_CHUNK = 128
_NCORES = 2
_NSUB = 16
_NW = _NCORES * _NSUB
_LANES = 16


def _sc_mesh():
    return plsc.VectorSubcoreMesh(core_axis_name="c", subcore_axis_name="s")


def _sc_hist(dst2d, n_pad):
    nchunks = dst2d.shape[0]
    cpw = nchunks // _NW
    sl = n_pad // _NSUB

    @functools.partial(
        pl.kernel,
        out_type=jax.ShapeDtypeStruct((_NCORES, n_pad, _LANES), jnp.float32),
        mesh=_sc_mesh(),
        scratch_types=[
            pltpu.VMEM_SHARED((n_pad, _LANES), jnp.float32),
            pltpu.VMEM((_CHUNK,), jnp.int32),
            pltpu.VMEM((_CHUNK,), jnp.int32),
            pltpu.VMEM((_CHUNK, _LANES), jnp.float32),
            pltpu.VMEM((_CHUNK, _LANES), jnp.float32),
            pltpu.SemaphoreType.DMA,
            pltpu.SemaphoreType.DMA,
        ],
    )
    def hist_kernel(dst_hbm, out_hbm, acc, idx_v, idx_w, ones_v, zero_v,
                    s0, s1):
        c = lax.axis_index("c")
        s = lax.axis_index("s")
        w = s * _NCORES + c

        @pl.loop(0, _CHUNK)
        def _(r):
            ones_v[r, :] = jnp.ones((_LANES,), jnp.float32)
            zero_v[r, :] = jnp.zeros((_LANES,), jnp.float32)

        @pl.loop(0, sl // _CHUNK)
        def _(b):
            pltpu.sync_copy(zero_v, acc.at[pl.ds(s * sl + b * _CHUNK, _CHUNK)])

        plsc.subcore_barrier()

        def load_idx(k, buf, sem):
            pltpu.async_copy(dst_hbm.at[w * cpw + k], buf, sem)

        def wait_idx(k, buf, sem):
            pltpu.make_async_copy(dst_hbm.at[w * cpw + k], buf, sem).wait()

        def load_idx(k, buf, sem):
            pltpu.async_copy(dst_hbm.at[w * cpw + k], buf, sem)

        def wait_idx(k, buf, sem):
            pltpu.make_async_copy(dst_hbm.at[w * cpw + k], buf, sem).wait()

        load_idx(0, idx_v, s0)
        load_idx(1, idx_w, s1)

        @pl.loop(0, cpw - 2, step=2)
        def _(k):
            wait_idx(k, idx_v, s0)
            pltpu.sync_copy(ones_v, acc.at[idx_v], add=True)
            load_idx(k + 2, idx_v, s0)
            wait_idx(k + 1, idx_w, s1)
            pltpu.sync_copy(ones_v, acc.at[idx_w], add=True)
            load_idx(k + 3, idx_w, s1)

        wait_idx(cpw - 2, idx_v, s0)
        pltpu.sync_copy(ones_v, acc.at[idx_v], add=True)
        wait_idx(cpw - 1, idx_w, s1)
        pltpu.sync_copy(ones_v, acc.at[idx_w], add=True)

        plsc.subcore_barrier()

        @pl.loop(0, sl // _CHUNK)
        def _(b):
            r0 = s * sl + b * _CHUNK
            pltpu.sync_copy(acc.at[pl.ds(r0, _CHUNK)],
                            out_hbm.at[c, pl.ds(r0, _CHUNK)])

    return hist_kernel(dst2d)


def _sc_agg(y_pad, src2d, dst2d):
    n_pad, d = y_pad.shape
    nchunks = src2d.shape[0]
    cpw = nchunks // _NW
    sl = n_pad // _NSUB
    nbuf = 2
    gdist = nbuf - 1
    assert cpw % nbuf == 0 and cpw >= 2 * nbuf

    @functools.partial(
        pl.kernel,
        out_type=jax.ShapeDtypeStruct((_NCORES, n_pad, d), jnp.float32),
        mesh=_sc_mesh(),
        compiler_params=pltpu.CompilerParams(use_tc_tiling_on_sc=False),
        scratch_types=[
            pltpu.VMEM_SHARED((n_pad, d), jnp.float32),
            [pltpu.VMEM((_CHUNK,), jnp.int32)] * nbuf,
            [pltpu.VMEM((_CHUNK,), jnp.int32)] * nbuf,
            [pltpu.VMEM((_CHUNK, d), jnp.float32)] * nbuf,
            pltpu.VMEM((_LANES, d), jnp.float32),
            [pltpu.SemaphoreType.DMA] * nbuf,
            [pltpu.SemaphoreType.DMA] * nbuf,
        ],
    )
    def agg_kernel(y_hbm, src_hbm, dst_hbm, out_hbm, acc,
                   si, di, rows, zero_v, gsem, isem):
        c = lax.axis_index("c")
        s = lax.axis_index("s")
        w = s * _NCORES + c
        first = w * cpw

        @pl.loop(0, _LANES)
        def _(r):
            @pl.loop(0, d // _LANES)
            def _(q):
                zero_v[r, pl.ds(q * _LANES, _LANES)] = jnp.zeros(
                    (_LANES,), jnp.float32)

        @pl.loop(0, sl // _LANES)
        def _(b):
            pltpu.sync_copy(zero_v, acc.at[pl.ds(s * sl + b * _LANES, _LANES)])

        plsc.subcore_barrier()

        def start_idx(k, b):
            pltpu.async_copy(src_hbm.at[first + k], si[b], isem[b])
            pltpu.async_copy(dst_hbm.at[first + k], di[b], isem[b])

        def wait_idx(k, b):
            pltpu.make_async_copy(src_hbm.at[first + k], si[b], isem[b]).wait()
            pltpu.make_async_copy(dst_hbm.at[first + k], di[b], isem[b]).wait()

        def start_gather(b):
            pltpu.async_copy(y_hbm.at[si[b]], rows[b], gsem[b])

        def wait_gather(b):
            pltpu.make_async_copy(y_hbm.at[si[b]], rows[b], gsem[b]).wait()

        def body(k, a, b):
            wait_gather(a)
            wait_idx(k + 1, b)
            start_gather(b)
            pltpu.sync_copy(rows[a], acc.at[di[a]], add=True)
            start_idx(k + 2, a)

        pltpu.sync_copy(src_hbm.at[first], si[0])
        pltpu.sync_copy(dst_hbm.at[first], di[0])
        start_gather(0)
        start_idx(1, 1)

        @pl.loop(0, cpw - 2, step=2)
        def _(k):
            body(k, 0, 1)
            body(k + 1, 1, 0)

        wait_gather(0)
        wait_idx(cpw - 1, 1)
        start_gather(1)
        pltpu.sync_copy(rows[0], acc.at[di[0]], add=True)
        wait_gather(1)
        pltpu.sync_copy(rows[1], acc.at[di[1]], add=True)

        plsc.subcore_barrier()

        @pl.loop(0, sl // _CHUNK)
        def _(b):
            r0 = s * sl + b * _CHUNK
            pltpu.sync_copy(acc.at[pl.ds(r0, _CHUNK)],
                            out_hbm.at[c, pl.ds(r0, _CHUNK)])

    return agg_kernel(y_pad, src2d, dst2d)


_BR = 1024


def _tc_mm_scale(x_pad, w, hist, n):
    n_pad, din = x_pad.shape
    dout = w.shape[1]

    def body(x_ref, w_ref, h_ref, y_ref, di_ref):
        i = pl.program_id(0)
        deg = h_ref[0, :, 0:1] + h_ref[1, :, 0:1] + 1.0
        rid = lax.broadcasted_iota(jnp.int32, (_BR, 1), 0) + i * _BR
        dinv = jnp.where(rid < n, lax.rsqrt(deg), 0.0)
        di_ref[...] = dinv
        xw = jnp.dot(x_ref[...], w_ref[...],
                     preferred_element_type=jnp.float32,
                     precision=lax.Precision.HIGHEST)
        y_ref[...] = xw * dinv

    return pl.pallas_call(
        body,
        grid=(n_pad // _BR,),
        in_specs=[pl.BlockSpec((_BR, din), lambda i: (i, 0)),
                  pl.BlockSpec((din, dout), lambda i: (0, 0)),
                  pl.BlockSpec((_NCORES, _BR, _LANES), lambda i: (0, i, 0))],
        out_specs=[pl.BlockSpec((_BR, dout), lambda i: (i, 0)),
                   pl.BlockSpec((_BR, 1), lambda i: (i, 0))],
        out_shape=[jax.ShapeDtypeStruct((n_pad, dout), jnp.float32),
                   jax.ShapeDtypeStruct((n_pad, 1), jnp.float32)],
    )(x_pad, w, hist)


def _tc_combine(parts, y, dinv, b, w2):
    n_pad, d = y.shape
    dout = w2.shape[1]

    def body(p_ref, y_ref, di_ref, b_ref, w_ref, o_ref):
        di = di_ref[...]
        h = jnp.maximum(di * (p_ref[0] + p_ref[1] + y_ref[...]) + b_ref[...],
                        0.0)
        o_ref[...] = di * jnp.dot(h, w_ref[...],
                                  preferred_element_type=jnp.float32,
                                  precision=lax.Precision.HIGHEST)

    return pl.pallas_call(
        body,
        grid=(n_pad // _BR,),
        in_specs=[pl.BlockSpec((_NCORES, _BR, d), lambda i: (0, i, 0)),
                  pl.BlockSpec((_BR, d), lambda i: (i, 0)),
                  pl.BlockSpec((_BR, 1), lambda i: (i, 0)),
                  pl.BlockSpec((1, d), lambda i: (0, 0)),
                  pl.BlockSpec((d, dout), lambda i: (0, 0))],
        out_specs=pl.BlockSpec((_BR, dout), lambda i: (i, 0)),
        out_shape=jax.ShapeDtypeStruct((n_pad, dout), jnp.float32),
    )(parts, y, dinv, b, w2)


def _tc_final(parts, y, dinv, b, n):
    n_pad, d = y.shape
    br = 2000
    assert n % br == 0

    def body(p_ref, y_ref, di_ref, b_ref, o_ref):
        o_ref[...] = (di_ref[...] * (p_ref[0] + p_ref[1] + y_ref[...])
                      + b_ref[...])

    return pl.pallas_call(
        body,
        grid=(n // br,),
        in_specs=[pl.BlockSpec((_NCORES, br, d), lambda i: (0, i, 0)),
                  pl.BlockSpec((br, d), lambda i: (i, 0)),
                  pl.BlockSpec((br, 1), lambda i: (i, 0)),
                  pl.BlockSpec((1, d), lambda i: (0, 0))],
        out_specs=pl.BlockSpec((br, d), lambda i: (i, 0)),
        out_shape=jax.ShapeDtypeStruct((n, d), jnp.float32),
    )(parts, y, dinv, b)


def kernel(x, edge_index, W1, b1, W2, b2):
    n, din = x.shape
    e = edge_index.shape[1]

    n_pad = ((n + _NSUB * _CHUNK - 1) // (_NSUB * _CHUNK)) * (_NSUB * _CHUNK)
    step = _NW * _CHUNK * 2
    e_pad = ((e + step - 1) // step) * step

    src = edge_index[0].astype(jnp.int32)
    dst = edge_index[1].astype(jnp.int32)
    spare = n_pad - n
    fill = n + jnp.arange(e_pad - e, dtype=jnp.int32) % spare
    src2d = jnp.concatenate([src, fill]).reshape(-1, _CHUNK)
    dst2d = jnp.concatenate([dst, fill]).reshape(-1, _CHUNK)
    x_pad = jnp.pad(x, ((0, n_pad - n), (0, 0)))
    b1r = b1.reshape(1, -1)
    b2r = b2.reshape(1, -1)

    hist = _sc_hist(dst2d, n_pad)
    y1, dinv = _tc_mm_scale(x_pad, W1, hist, n)
    parts1 = _sc_agg(y1, src2d, dst2d)
    y2 = _tc_combine(parts1, y1, dinv, b1r, W2)
    parts2 = _sc_agg(y2, src2d, dst2d)
    return _tc_final(parts2, y2, dinv, b2r, n)
